# Initial kernel scaffold; baseline (speedup 1.0000x reference)
import jax
import jax.numpy as jnp
from jax.experimental import pallas as pl


def kernel(literal_feat, clause_feat, W_l2c, b_l2c, W_c2l, b_c2l, lit_idx, clause_idx):
    raise NotImplementedError("write your pallas kernel here")


# speedup vs baseline: 4.2189x; 4.2189x over previous
import functools

import jax
import jax.numpy as jnp
from jax import lax
from jax.experimental import pallas as pl
from jax.experimental.pallas import tpu as pltpu
from jax.experimental.pallas import tpu_sc as plsc

_NS = 16
_GRP = 128
_GRP_PER_CHUNK = 4
_CHUNK = _GRP * _GRP_PER_CHUNK


def _make_fused(n1_pad, n2a, n2b, n_acc, d, gpw, n_src):
  assert n1_pad % (_NS * 8) == 0 and n_acc % (_NS * 8) == 0
  assert n2a % (_NS * 8) == 0 and n2b % (_NS * 8) == 0
  assert gpw % _GRP_PER_CHUNK == 0
  n_chunks = gpw // _GRP_PER_CHUNK
  r1 = n1_pad // _NS
  ra = n_acc // _NS
  sub1 = r1 // 2
  mesh = plsc.VectorSubcoreMesh(core_axis_name="c", subcore_axis_name="s",
                                num_cores=1)

  @functools.partial(
      pl.kernel,
      out_type=(jax.ShapeDtypeStruct((n1_pad, d), jnp.float32),
                jax.ShapeDtypeStruct((n2a + n2b, d), jnp.float32)),
      mesh=mesh,
      scratch_types=[
          pltpu.VMEM((gpw // 2, _GRP), jnp.int32),
          pltpu.VMEM((gpw // 2, _GRP), jnp.int32),
          pltpu.VMEM((_CHUNK, d), jnp.float32),
          pltpu.VMEM((d,), jnp.float32),
          pltpu.VMEM_SHARED((n_acc, d), jnp.float32),
          pltpu.SemaphoreType.DMA,
      ],
  )
  def fused(lit_t, cfa, bias, sidx1, didx1, sidx2, didx2,
            zeros, table2, q, sidx_v, didx_v, rows_v, bias_v, acc, sem):
    s = lax.axis_index("s")

    def zero_acc():
      pltpu.sync_copy(zeros.at[pl.ds(s * ra, ra)], acc.at[pl.ds(s * ra, ra)])

    def remap(sub, half):
      base_g = (s * gpw + half * (gpw // 2)) * _GRP
      lanes = lax.iota(jnp.int32, 16)

      def row_body(r, carry):
        for k in range(_GRP // 16):
          sl = pl.ds(k * 16, 16)
          sv = sidx_v[r, sl]
          dv = didx_v[r, sl]
          g = base_g + r * _GRP + k * 16 + lanes
          spread = lax.rem(g, jnp.int32(n_src))
          if sub == 0:
            m = dv < n2a
            trash = n2a + lax.bitwise_and(g, jnp.int32(255))
            dnew = dv
          else:
            m = dv >= n2a
            trash = n2b + lax.bitwise_and(g, jnp.int32(255))
            dnew = dv - n2a
          sidx_v[r, sl] = jnp.where(m, sv, spread)
          didx_v[r, sl] = jnp.where(m, dnew, trash)
        return carry

      lax.fori_loop(0, gpw // 2, row_body, 0)

    def seg_sum_round(src_idx, dst_idx, table, sub=None):
      for half in range(2):
        base_grp = s * gpw + half * (gpw // 2)
        pltpu.sync_copy(src_idx.at[pl.ds(base_grp, gpw // 2)], sidx_v)
        pltpu.sync_copy(dst_idx.at[pl.ds(base_grp, gpw // 2)], didx_v)
        if sub is not None:
          remap(sub, half)

        def chunk_body(j, carry):
          base = j * _GRP_PER_CHUNK
          copies = []
          for i in range(_GRP_PER_CHUNK):
            copies.append(
                pltpu.async_copy(table.at[sidx_v.at[base + i]],
                                 rows_v.at[pl.ds(i * _GRP, _GRP)], sem))
          for cp in copies:
            cp.wait()
          for i in range(_GRP_PER_CHUNK):
            pltpu.sync_copy(rows_v.at[pl.ds(i * _GRP, _GRP)],
                            acc.at[didx_v.at[base + i]], add=True)
          return carry

        lax.fori_loop(0, n_chunks // 2, chunk_body, 0)
      plsc.subcore_barrier()

    zero_acc()
    pltpu.sync_copy(bias, bias_v)
    plsc.subcore_barrier()
    seg_sum_round(sidx1, didx1, lit_t)

    bv = [bias_v[pl.ds(k * 16, 16)] for k in range(d // 16)]
    for half in range(2):
      base_row = s * r1 + half * sub1
      pltpu.sync_copy(acc.at[pl.ds(base_row, sub1)], rows_v.at[pl.ds(0, sub1)])
      pltpu.sync_copy(cfa.at[pl.ds(base_row, sub1)],
                      rows_v.at[pl.ds(sub1, sub1)])

      def row_body(r, carry):
        for k in range(d // 16):
          x = rows_v[r, pl.ds(k * 16, 16)]
          a = rows_v[sub1 + r, pl.ds(k * 16, 16)]
          rows_v[r, pl.ds(k * 16, 16)] = jnp.maximum(x + bv[k], 0.0) + a
        return carry

      lax.fori_loop(0, sub1, row_body, 0)
      pltpu.sync_copy(rows_v.at[pl.ds(0, sub1)],
                      table2.at[pl.ds(base_row, sub1)])

    plsc.subcore_barrier()
    zero_acc()
    plsc.subcore_barrier()
    seg_sum_round(sidx2, didx2, table2, sub=0)
    pltpu.sync_copy(acc.at[pl.ds(s * (n2a // _NS), n2a // _NS)],
                    q.at[pl.ds(s * (n2a // _NS), n2a // _NS)])
    plsc.subcore_barrier()

    zero_acc()
    plsc.subcore_barrier()
    seg_sum_round(sidx2, didx2, table2, sub=1)
    pltpu.sync_copy(acc.at[pl.ds(s * (n2b // _NS), n2b // _NS)],
                    q.at[pl.ds(n2a + s * (n2b // _NS), n2b // _NS)])

  return fused


def _mm_body(x_ref, w_ref, o_ref):
  o_ref[...] = jnp.dot(x_ref[...], w_ref[...].T,
                       preferred_element_type=jnp.float32)


def _outer_body(cf_ref, a_ref, o_ref):
  o_ref[...] = cf_ref[...] * a_ref[...]


def _mm_relu_body(x_ref, w_ref, b_ref, o_ref):
  y = jnp.dot(x_ref[...], w_ref[...].T, preferred_element_type=jnp.float32)
  o_ref[...] = jnp.maximum(y + b_ref[...], 0.0)


def kernel(literal_feat, clause_feat, W_l2c, b_l2c, W_c2l, b_c2l, lit_idx,
           clause_idx):
  n_lit, in_size = literal_feat.shape
  n_clause = clause_feat.shape[0]
  e = lit_idx.shape[0]
  out_size = W_c2l.shape[0]
  d = in_size

  gpw = -(-e // (_NS * _GRP * _GRP_PER_CHUNK)) * _GRP_PER_CHUNK
  e_pad = _NS * gpw * _GRP
  npad = e_pad - e
  n1_pad = -(-(n_clause + 16) // 128) * 128
  n2a = n1_pad
  n2b = -(-(n_lit + 16 - n2a) // 128) * 128
  n_acc = n2a + 256

  ar_pad = jnp.arange(npad, dtype=jnp.int32)
  lit_i = lit_idx.astype(jnp.int32)
  cla_i = clause_idx.astype(jnp.int32)
  sidx1 = jnp.concatenate([lit_i, ar_pad % n_lit]).reshape(-1, _GRP)
  didx1 = jnp.concatenate([cla_i, n_clause + ar_pad % 16]).reshape(-1, _GRP)
  sidx2 = jnp.concatenate([cla_i, ar_pad % n_clause]).reshape(-1, _GRP)
  didx2 = jnp.concatenate([lit_i, n_lit + ar_pad % 16]).reshape(-1, _GRP)
  zeros = jnp.zeros((n_acc, d), jnp.float32)

  w2 = W_c2l[:, :in_size]
  wcol = W_c2l[:, in_size]
  alpha = jnp.linalg.solve(w2, wcol)
  alpha = alpha + jnp.linalg.solve(w2, wcol - w2 @ alpha)

  blk = 1000
  lit_t = pl.pallas_call(
      _mm_body,
      grid=(n_lit // blk,),
      in_specs=[pl.BlockSpec((blk, in_size), lambda i: (i, 0)),
                pl.BlockSpec((out_size, in_size), lambda i: (0, 0))],
      out_specs=pl.BlockSpec((blk, out_size), lambda i: (i, 0)),
      out_shape=jax.ShapeDtypeStruct((n_lit, out_size), jnp.float32),
  )(literal_feat, W_l2c)

  cf_pad = jnp.concatenate(
      [clause_feat.astype(jnp.float32),
       jnp.zeros((n1_pad - n_clause, 1), jnp.float32)])
  cfa = pl.pallas_call(
      _outer_body,
      grid=(1,),
      in_specs=[pl.BlockSpec((n1_pad, 1), lambda i: (0, 0)),
                pl.BlockSpec((1, d), lambda i: (0, 0))],
      out_specs=pl.BlockSpec((n1_pad, d), lambda i: (0, 0)),
      out_shape=jax.ShapeDtypeStruct((n1_pad, d), jnp.float32),
  )(cf_pad, alpha.reshape(1, -1))

  fused = _make_fused(n1_pad, n2a, n2b, n_acc, d, gpw, n_clause)
  _, q = fused(lit_t, cfa, b_l2c.astype(jnp.float32), sidx1, didx1,
               sidx2, didx2, zeros)

  lembs = pl.pallas_call(
      _mm_relu_body,
      grid=(n_lit // blk,),
      in_specs=[pl.BlockSpec((blk, d), lambda i: (i, 0)),
                pl.BlockSpec((out_size, d), lambda i: (0, 0)),
                pl.BlockSpec((1, out_size), lambda i: (0, 0))],
      out_specs=pl.BlockSpec((blk, out_size), lambda i: (i, 0)),
      out_shape=jax.ShapeDtypeStruct((n_lit, out_size), jnp.float32),
  )(q[:n_lit], w2, b_c2l.reshape(1, -1))

  return lembs


# speedup vs baseline: 5.4643x; 1.2952x over previous
import functools

import jax
import jax.numpy as jnp
from jax import lax
from jax.experimental import pallas as pl
from jax.experimental.pallas import tpu as pltpu
from jax.experimental.pallas import tpu_sc as plsc

_NS = 16
_GRP = 128
_GRP_PER_CHUNK = 4
_GPB = 2
_CHUNK = _GRP * _GRP_PER_CHUNK


def _make_fused(n1_pad, n2a, n2b, n_acc, d, gpw, n_src):
  assert n1_pad % (_NS * 8) == 0 and n_acc % (_NS * 8) == 0
  assert n2a % (_NS * 8) == 0 and n2b % (_NS * 8) == 0
  assert gpw % _GRP_PER_CHUNK == 0
  n_chunks = gpw // _GRP_PER_CHUNK
  r1 = n1_pad // _NS
  ra = n_acc // _NS
  sub1 = r1 // 2
  mesh = plsc.VectorSubcoreMesh(core_axis_name="c", subcore_axis_name="s",
                                num_cores=1)

  @functools.partial(
      pl.kernel,
      out_type=(jax.ShapeDtypeStruct((n1_pad, d), jnp.float32),
                jax.ShapeDtypeStruct((n2a + n2b, d), jnp.float32)),
      mesh=mesh,
      scratch_types=[
          pltpu.VMEM((gpw // 2, _GRP), jnp.int32),
          pltpu.VMEM((gpw // 2, _GRP), jnp.int32),
          pltpu.VMEM((_CHUNK, d), jnp.float32),
          pltpu.VMEM((d,), jnp.float32),
          pltpu.VMEM_SHARED((n_acc, d), jnp.float32),
          pltpu.SemaphoreType.DMA,
          pltpu.SemaphoreType.DMA,
          pltpu.SemaphoreType.DMA,
          pltpu.SemaphoreType.DMA,
      ],
  )
  def fused(lit_t, cfa, bias, sidx1, didx1, sidx2, didx2,
            zeros, table2, q, sidx_v, didx_v, rows_v, bias_v, acc,
            sg0, sg1, ss0, ss1):
    s = lax.axis_index("s")

    def zero_acc():
      pltpu.sync_copy(zeros.at[pl.ds(s * ra, ra)], acc.at[pl.ds(s * ra, ra)])

    def remap(sub, half):
      base_g = (s * gpw + half * (gpw // 2)) * _GRP
      lanes = lax.iota(jnp.int32, 16)

      def row_body(r, carry):
        for k in range(_GRP // 16):
          sl = pl.ds(k * 16, 16)
          sv = sidx_v[r, sl]
          dv = didx_v[r, sl]
          g = base_g + r * _GRP + k * 16 + lanes
          spread = lax.rem(g, jnp.int32(n_src))
          if sub == 0:
            m = dv < n2a
            trash = n2a + lax.bitwise_and(g, jnp.int32(255))
            dnew = dv
          else:
            m = dv >= n2a
            trash = n2b + lax.bitwise_and(g, jnp.int32(255))
            dnew = dv - n2a
          sidx_v[r, sl] = jnp.where(m, sv, spread)
          didx_v[r, sl] = jnp.where(m, dnew, trash)
        return carry

      lax.fori_loop(0, gpw // 2, row_body, 0)

    def seg_sum_round(src_idx, dst_idx, table, sub=None):
      for half in range(2):
        base_grp = s * gpw + half * (gpw // 2)
        pltpu.sync_copy(src_idx.at[pl.ds(base_grp, gpw // 2)], sidx_v)
        pltpu.sync_copy(dst_idx.at[pl.ds(base_grp, gpw // 2)], didx_v)
        if sub is not None:
          remap(sub, half)

        gsem = (sg0, sg1)
        ssem = (ss0, ss1)

        def issue_g(c, b):
          for i in range(_GPB):
            pltpu.async_copy(table.at[sidx_v.at[c * _GPB + i]],
                             rows_v.at[pl.ds((b * _GPB + i) * _GRP, _GRP)],
                             gsem[b])

        def wait_g(b):
          for i in range(_GPB):
            pltpu.make_async_copy(
                table.at[pl.ds(0, _GRP)],
                rows_v.at[pl.ds((b * _GPB + i) * _GRP, _GRP)],
                gsem[b]).wait()

        def scat(c, b):
          for i in range(_GPB):
            pltpu.sync_copy(rows_v.at[pl.ds((b * _GPB + i) * _GRP, _GRP)],
                            acc.at[didx_v.at[c * _GPB + i]], add=True)

        def wait_s(b):
          pass

        nch = n_chunks
        issue_g(0, 0)
        issue_g(1, 1)

        def chunk_body(jj, carry):
          c0 = jj * 2
          wait_g(0)
          scat(c0, 0)
          wait_s(0)
          issue_g(c0 + 2, 0)
          wait_g(1)
          scat(c0 + 1, 1)
          wait_s(1)
          issue_g(c0 + 3, 1)
          return carry

        lax.fori_loop(0, nch // 2 - 1, chunk_body, 0)
        wait_g(0)
        scat(nch - 2, 0)
        wait_g(1)
        scat(nch - 1, 1)
        wait_s(0)
        wait_s(1)
      plsc.subcore_barrier()

    zero_acc()
    pltpu.sync_copy(bias, bias_v)
    plsc.subcore_barrier()
    seg_sum_round(sidx1, didx1, lit_t)

    bv = [bias_v[pl.ds(k * 16, 16)] for k in range(d // 16)]
    for half in range(2):
      base_row = s * r1 + half * sub1
      pltpu.sync_copy(acc.at[pl.ds(base_row, sub1)], rows_v.at[pl.ds(0, sub1)])
      pltpu.sync_copy(cfa.at[pl.ds(base_row, sub1)],
                      rows_v.at[pl.ds(sub1, sub1)])

      def row_body(r, carry):
        for k in range(d // 16):
          x = rows_v[r, pl.ds(k * 16, 16)]
          a = rows_v[sub1 + r, pl.ds(k * 16, 16)]
          rows_v[r, pl.ds(k * 16, 16)] = jnp.maximum(x + bv[k], 0.0) + a
        return carry

      lax.fori_loop(0, sub1, row_body, 0)
      pltpu.sync_copy(rows_v.at[pl.ds(0, sub1)],
                      table2.at[pl.ds(base_row, sub1)])

    plsc.subcore_barrier()
    zero_acc()
    plsc.subcore_barrier()
    seg_sum_round(sidx2, didx2, table2, sub=0)
    pltpu.sync_copy(acc.at[pl.ds(s * (n2a // _NS), n2a // _NS)],
                    q.at[pl.ds(s * (n2a // _NS), n2a // _NS)])
    plsc.subcore_barrier()

    zero_acc()
    plsc.subcore_barrier()
    seg_sum_round(sidx2, didx2, table2, sub=1)
    pltpu.sync_copy(acc.at[pl.ds(s * (n2b // _NS), n2b // _NS)],
                    q.at[pl.ds(n2a + s * (n2b // _NS), n2b // _NS)])

  return fused


def _mm_body(x_ref, w_ref, o_ref):
  o_ref[...] = jnp.dot(x_ref[...], w_ref[...].T,
                       preferred_element_type=jnp.float32)


def _outer_body(cf_ref, a_ref, o_ref):
  o_ref[...] = cf_ref[...] * a_ref[...]


def _mm_relu_body(x_ref, w_ref, b_ref, o_ref):
  y = jnp.dot(x_ref[...], w_ref[...].T, preferred_element_type=jnp.float32)
  o_ref[...] = jnp.maximum(y + b_ref[...], 0.0)


def kernel(literal_feat, clause_feat, W_l2c, b_l2c, W_c2l, b_c2l, lit_idx,
           clause_idx):
  n_lit, in_size = literal_feat.shape
  n_clause = clause_feat.shape[0]
  e = lit_idx.shape[0]
  out_size = W_c2l.shape[0]
  d = in_size

  gpw = -(-e // (_NS * _GRP * _GRP_PER_CHUNK)) * _GRP_PER_CHUNK
  e_pad = _NS * gpw * _GRP
  npad = e_pad - e
  n1_pad = -(-(n_clause + 16) // 128) * 128
  n2a = n1_pad
  n2b = -(-(n_lit + 16 - n2a) // 128) * 128
  n_acc = n2a + 256

  ar_pad = jnp.arange(npad, dtype=jnp.int32)
  lit_i = lit_idx.astype(jnp.int32)
  cla_i = clause_idx.astype(jnp.int32)
  sidx1 = jnp.concatenate([lit_i, ar_pad % n_lit]).reshape(-1, _GRP)
  didx1 = jnp.concatenate([cla_i, n_clause + ar_pad % 16]).reshape(-1, _GRP)
  sidx2 = jnp.concatenate([cla_i, ar_pad % n_clause]).reshape(-1, _GRP)
  didx2 = jnp.concatenate([lit_i, n_lit + ar_pad % 16]).reshape(-1, _GRP)
  zeros = jnp.zeros((n_acc, d), jnp.float32)

  w2 = W_c2l[:, :in_size]
  wcol = W_c2l[:, in_size]
  alpha = jnp.linalg.solve(w2, wcol)
  alpha = alpha + jnp.linalg.solve(w2, wcol - w2 @ alpha)

  blk = 1000
  lit_t = pl.pallas_call(
      _mm_body,
      grid=(n_lit // blk,),
      in_specs=[pl.BlockSpec((blk, in_size), lambda i: (i, 0)),
                pl.BlockSpec((out_size, in_size), lambda i: (0, 0))],
      out_specs=pl.BlockSpec((blk, out_size), lambda i: (i, 0)),
      out_shape=jax.ShapeDtypeStruct((n_lit, out_size), jnp.float32),
  )(literal_feat, W_l2c)

  cf_pad = jnp.concatenate(
      [clause_feat.astype(jnp.float32),
       jnp.zeros((n1_pad - n_clause, 1), jnp.float32)])
  cfa = pl.pallas_call(
      _outer_body,
      grid=(1,),
      in_specs=[pl.BlockSpec((n1_pad, 1), lambda i: (0, 0)),
                pl.BlockSpec((1, d), lambda i: (0, 0))],
      out_specs=pl.BlockSpec((n1_pad, d), lambda i: (0, 0)),
      out_shape=jax.ShapeDtypeStruct((n1_pad, d), jnp.float32),
  )(cf_pad, alpha.reshape(1, -1))

  fused = _make_fused(n1_pad, n2a, n2b, n_acc, d, gpw, n_clause)
  _, q = fused(lit_t, cfa, b_l2c.astype(jnp.float32), sidx1, didx1,
               sidx2, didx2, zeros)

  lembs = pl.pallas_call(
      _mm_relu_body,
      grid=(n_lit // blk,),
      in_specs=[pl.BlockSpec((blk, d), lambda i: (i, 0)),
                pl.BlockSpec((out_size, d), lambda i: (0, 0)),
                pl.BlockSpec((1, out_size), lambda i: (0, 0))],
      out_specs=pl.BlockSpec((blk, out_size), lambda i: (i, 0)),
      out_shape=jax.ShapeDtypeStruct((n_lit, out_size), jnp.float32),
  )(q[:n_lit], w2, b_c2l.reshape(1, -1))

  return lembs


# speedup vs baseline: 5.4704x; 1.0011x over previous
import functools

import jax
import jax.numpy as jnp
from jax import lax
from jax.experimental import pallas as pl
from jax.experimental.pallas import tpu as pltpu
from jax.experimental.pallas import tpu_sc as plsc

_NS = 16
_GRP = 128
_GRP_PER_CHUNK = 4
_GPB = 2
_CHUNK = _GRP * _GRP_PER_CHUNK


def _make_fused(n1_pad, n2a, n2b, n_acc, d, gpw, n_src):
  assert n1_pad % (_NS * 8) == 0 and n_acc % (_NS * 8) == 0
  assert n2a % (_NS * 8) == 0 and n2b % (_NS * 8) == 0
  assert gpw % _GRP_PER_CHUNK == 0
  n_chunks = gpw // _GRP_PER_CHUNK
  r1 = n1_pad // _NS
  ra = n_acc // _NS
  sub1 = r1 // 2
  mesh = plsc.VectorSubcoreMesh(core_axis_name="c", subcore_axis_name="s",
                                num_cores=1)

  @functools.partial(
      pl.kernel,
      out_type=(jax.ShapeDtypeStruct((n1_pad, d), jnp.float32),
                jax.ShapeDtypeStruct((n2a + n2b, d), jnp.float32)),
      mesh=mesh,
      scratch_types=[
          pltpu.VMEM((gpw // 2, _GRP), jnp.int32),
          pltpu.VMEM((gpw // 2, _GRP), jnp.int32),
          pltpu.VMEM((_CHUNK, d), jnp.float32),
          pltpu.VMEM((d,), jnp.float32),
          pltpu.VMEM_SHARED((n_acc, d), jnp.float32),
          pltpu.SemaphoreType.DMA,
          pltpu.SemaphoreType.DMA,
          pltpu.SemaphoreType.DMA,
          pltpu.SemaphoreType.DMA,
      ],
  )
  def fused(lit_t, cfa, bias, sidx1, didx1, sidx2, didx2,
            zeros, table2, q, sidx_v, didx_v, rows_v, bias_v, acc,
            sg0, sg1, ss0, ss1):
    s = lax.axis_index("s")

    def zero_acc():
      pltpu.sync_copy(zeros.at[pl.ds(s * ra, ra)], acc.at[pl.ds(s * ra, ra)])

    def remap(sub, half):
      base_g = (s * gpw + half * (gpw // 2)) * _GRP
      lanes = lax.iota(jnp.int32, 16)

      def row_body(r, carry):
        for k in range(_GRP // 16):
          sl = pl.ds(k * 16, 16)
          sv = sidx_v[r, sl]
          dv = didx_v[r, sl]
          g = base_g + r * _GRP + k * 16 + lanes
          spread = lax.rem(g, jnp.int32(n_src))
          if sub == 0:
            m = dv < n2a
            trash = n2a + lax.bitwise_and(g, jnp.int32(255))
            dnew = dv
          else:
            m = dv >= n2a
            trash = n2b + lax.bitwise_and(g, jnp.int32(255))
            dnew = dv - n2a
          sidx_v[r, sl] = jnp.where(m, sv, spread)
          didx_v[r, sl] = jnp.where(m, dnew, trash)
        return carry

      lax.fori_loop(0, gpw // 2, row_body, 0)

    def seg_sum_round(src_idx, dst_idx, table, sub=None):
      for half in range(2):
        base_grp = s * gpw + half * (gpw // 2)
        pltpu.sync_copy(src_idx.at[pl.ds(base_grp, gpw // 2)], sidx_v)
        pltpu.sync_copy(dst_idx.at[pl.ds(base_grp, gpw // 2)], didx_v)
        if sub is not None:
          remap(sub, half)

        gsem = (sg0, sg1)

        def issue_g(c, b):
          for i in range(_GPB):
            pltpu.async_copy(table.at[sidx_v.at[c * _GPB + i]],
                             rows_v.at[pl.ds((b * _GPB + i) * _GRP, _GRP)],
                             gsem[b])

        def wait_g(b):
          for i in range(_GPB):
            pltpu.make_async_copy(
                table.at[pl.ds(0, _GRP)],
                rows_v.at[pl.ds((b * _GPB + i) * _GRP, _GRP)],
                gsem[b]).wait()

        def scat(c, b):
          for i in range(_GPB):
            pltpu.sync_copy(rows_v.at[pl.ds((b * _GPB + i) * _GRP, _GRP)],
                            acc.at[didx_v.at[c * _GPB + i]], add=True)

        nch = n_chunks
        issue_g(0, 0)
        issue_g(1, 1)

        def chunk_body(jj, carry):
          c0 = jj * 2
          wait_g(0)
          scat(c0, 0)
          issue_g(c0 + 2, 0)
          wait_g(1)
          scat(c0 + 1, 1)
          issue_g(c0 + 3, 1)
          return carry

        lax.fori_loop(0, nch // 2 - 1, chunk_body, 0)
        wait_g(0)
        scat(nch - 2, 0)
        wait_g(1)
        scat(nch - 1, 1)
      plsc.subcore_barrier()

    zero_acc()
    pltpu.sync_copy(bias, bias_v)
    plsc.subcore_barrier()
    seg_sum_round(sidx1, didx1, lit_t)

    bv = [bias_v[pl.ds(k * 16, 16)] for k in range(d // 16)]
    for half in range(2):
      base_row = s * r1 + half * sub1
      pltpu.sync_copy(acc.at[pl.ds(base_row, sub1)], rows_v.at[pl.ds(0, sub1)])
      pltpu.sync_copy(cfa.at[pl.ds(base_row, sub1)],
                      rows_v.at[pl.ds(sub1, sub1)])

      def row_body(r, carry):
        for k in range(d // 16):
          x = rows_v[r, pl.ds(k * 16, 16)]
          a = rows_v[sub1 + r, pl.ds(k * 16, 16)]
          rows_v[r, pl.ds(k * 16, 16)] = jnp.maximum(x + bv[k], 0.0) + a
        return carry

      lax.fori_loop(0, sub1, row_body, 0)
      pltpu.sync_copy(rows_v.at[pl.ds(0, sub1)],
                      table2.at[pl.ds(base_row, sub1)])

    plsc.subcore_barrier()
    zero_acc()
    plsc.subcore_barrier()
    seg_sum_round(sidx2, didx2, table2, sub=0)
    pltpu.sync_copy(acc.at[pl.ds(s * (n2a // _NS), n2a // _NS)],
                    q.at[pl.ds(s * (n2a // _NS), n2a // _NS)])
    plsc.subcore_barrier()

    zero_acc()
    plsc.subcore_barrier()
    seg_sum_round(sidx2, didx2, table2, sub=1)
    pltpu.sync_copy(acc.at[pl.ds(s * (n2b // _NS), n2b // _NS)],
                    q.at[pl.ds(n2a + s * (n2b // _NS), n2b // _NS)])

  return fused


def _mm_body(x_ref, w_ref, o_ref):
  o_ref[...] = jnp.dot(x_ref[...], w_ref[...].T,
                       preferred_element_type=jnp.float32)


def _outer_body(cf_ref, a_ref, o_ref):
  o_ref[...] = cf_ref[...] * a_ref[...]


def _mm_relu_body(x_ref, w_ref, b_ref, o_ref):
  y = jnp.dot(x_ref[...], w_ref[...].T, preferred_element_type=jnp.float32)
  o_ref[...] = jnp.maximum(y + b_ref[...], 0.0)


def kernel(literal_feat, clause_feat, W_l2c, b_l2c, W_c2l, b_c2l, lit_idx,
           clause_idx):
  n_lit, in_size = literal_feat.shape
  n_clause = clause_feat.shape[0]
  e = lit_idx.shape[0]
  out_size = W_c2l.shape[0]
  d = in_size

  gpw = -(-e // (_NS * _GRP * _GRP_PER_CHUNK)) * _GRP_PER_CHUNK
  e_pad = _NS * gpw * _GRP
  npad = e_pad - e
  n1_pad = -(-(n_clause + 16) // 128) * 128
  n2a = n1_pad
  n2b = -(-(n_lit + 16 - n2a) // 128) * 128
  n_acc = n2a + 256

  ar_pad = jnp.arange(npad, dtype=jnp.int32)
  lit_i = lit_idx.astype(jnp.int32)
  cla_i = clause_idx.astype(jnp.int32)
  sidx1 = jnp.concatenate([lit_i, ar_pad % n_lit]).reshape(-1, _GRP)
  didx1 = jnp.concatenate([cla_i, n_clause + ar_pad % 16]).reshape(-1, _GRP)
  sidx2 = jnp.concatenate([cla_i, ar_pad % n_clause]).reshape(-1, _GRP)
  didx2 = jnp.concatenate([lit_i, n_lit + ar_pad % 16]).reshape(-1, _GRP)
  zeros = jnp.zeros((n_acc, d), jnp.float32)

  w2 = W_c2l[:, :in_size]
  wcol = W_c2l[:, in_size]
  alpha = jnp.linalg.solve(w2, wcol)
  alpha = alpha + jnp.linalg.solve(w2, wcol - w2 @ alpha)

  blk = 1000
  lit_t = pl.pallas_call(
      _mm_body,
      grid=(n_lit // blk,),
      in_specs=[pl.BlockSpec((blk, in_size), lambda i: (i, 0)),
                pl.BlockSpec((out_size, in_size), lambda i: (0, 0))],
      out_specs=pl.BlockSpec((blk, out_size), lambda i: (i, 0)),
      out_shape=jax.ShapeDtypeStruct((n_lit, out_size), jnp.float32),
  )(literal_feat, W_l2c)

  cf_pad = jnp.concatenate(
      [clause_feat.astype(jnp.float32),
       jnp.zeros((n1_pad - n_clause, 1), jnp.float32)])
  cfa = pl.pallas_call(
      _outer_body,
      grid=(1,),
      in_specs=[pl.BlockSpec((n1_pad, 1), lambda i: (0, 0)),
                pl.BlockSpec((1, d), lambda i: (0, 0))],
      out_specs=pl.BlockSpec((n1_pad, d), lambda i: (0, 0)),
      out_shape=jax.ShapeDtypeStruct((n1_pad, d), jnp.float32),
  )(cf_pad, alpha.reshape(1, -1))

  fused = _make_fused(n1_pad, n2a, n2b, n_acc, d, gpw, n_clause)
  _, q = fused(lit_t, cfa, b_l2c.astype(jnp.float32), sidx1, didx1,
               sidx2, didx2, zeros)

  lembs = pl.pallas_call(
      _mm_relu_body,
      grid=(n_lit // blk,),
      in_specs=[pl.BlockSpec((blk, d), lambda i: (i, 0)),
                pl.BlockSpec((out_size, d), lambda i: (0, 0)),
                pl.BlockSpec((1, out_size), lambda i: (0, 0))],
      out_specs=pl.BlockSpec((blk, out_size), lambda i: (i, 0)),
      out_shape=jax.ShapeDtypeStruct((n_lit, out_size), jnp.float32),
  )(q[:n_lit], w2, b_c2l.reshape(1, -1))

  return lembs
